# baseline (speedup 1.0000x reference)
import functools

import jax
import jax.numpy as jnp
from jax import lax
from jax.experimental import pallas as pl
from jax.experimental.pallas import tpu as pltpu
from jax.experimental.pallas import tpu_sc as plsc

NC = 2
NS = 16
NW = NC * NS
LANE = 16

B = 4096
L = 200
EMB = 64
NLAB = 5

BPW = B // NW
NIDX = BPW * L
CA, CB = 104, 96
NVEC = EMB // LANE
INV_L = 1.0 / L


def _accum_chunk(buf, n, acc):
    def body(j, acc):
        out = list(acc)
        for u in range(4):
            r = 4 * j + u
            for k in range(NVEC):
                out[k] = out[k] + buf[r, pl.ds(k * LANE, LANE)]
        return tuple(out)

    return lax.fori_loop(0, n // 4, body, acc)


@functools.partial(
    pl.kernel,
    out_type=jax.ShapeDtypeStruct((B, EMB), jnp.float32),
    mesh=plsc.VectorSubcoreMesh(core_axis_name="c", subcore_axis_name="s"),
    compiler_params=pltpu.CompilerParams(use_tc_tiling_on_sc=False),
    scratch_types=[
        pltpu.VMEM((NIDX,), jnp.int32),
        pltpu.VMEM((BPW, EMB), jnp.float32),
        pltpu.VMEM((CA, EMB), jnp.float32),
        pltpu.VMEM((CB, EMB), jnp.float32),
        pltpu.VMEM((CA, EMB), jnp.float32),
        pltpu.VMEM((CB, EMB), jnp.float32),
        pltpu.SemaphoreType.DMA,
        pltpu.SemaphoreType.DMA,
        pltpu.SemaphoreType.DMA,
        pltpu.SemaphoreType.DMA,
    ],
)
def _pool_kernel(idx_hbm, table_hbm, out_hbm, idx_v, pooled_v,
                 buf_a0, buf_b0, buf_a1, buf_b1,
                 sem_a0, sem_b0, sem_a1, sem_b1):
    wid = lax.axis_index("s") * NC + lax.axis_index("c")

    pltpu.sync_copy(idx_hbm.at[pl.ds(wid * NIDX, NIDX)], idx_v)

    def fire(row, off, size, buf, sem):
        start = row * L + off
        pltpu.async_copy(table_hbm.at[idx_v.at[pl.ds(start, size)]], buf, sem)

    def wait(size, buf, sem):
        pltpu.make_async_copy(table_hbm.at[pl.ds(0, size)], buf, sem).wait()

    fire(0, 0, CA, buf_a0, sem_a0)
    fire(0, CA, CB, buf_b0, sem_b0)
    fire(1, 0, CA, buf_a1, sem_a1)
    fire(1, CA, CB, buf_b1, sem_b1)

    zero = jnp.zeros((LANE,), jnp.float32)

    def step(t, carry):
        del carry
        r0 = 2 * t
        r1 = r0 + 1
        n0 = (r0 + 2) & (BPW - 1)
        n1 = (r1 + 2) & (BPW - 1)

        acc = (zero, zero, zero, zero)
        wait(CA, buf_a0, sem_a0)
        acc = _accum_chunk(buf_a0, CA, acc)
        fire(n0, 0, CA, buf_a0, sem_a0)
        wait(CB, buf_b0, sem_b0)
        acc = _accum_chunk(buf_b0, CB, acc)
        fire(n0, CA, CB, buf_b0, sem_b0)
        for k in range(NVEC):
            pooled_v[r0, pl.ds(k * LANE, LANE)] = acc[k] * INV_L

        acc = (zero, zero, zero, zero)
        wait(CA, buf_a1, sem_a1)
        acc = _accum_chunk(buf_a1, CA, acc)
        fire(n1, 0, CA, buf_a1, sem_a1)
        wait(CB, buf_b1, sem_b1)
        acc = _accum_chunk(buf_b1, CB, acc)
        fire(n1, CA, CB, buf_b1, sem_b1)
        for k in range(NVEC):
            pooled_v[r1, pl.ds(k * LANE, LANE)] = acc[k] * INV_L

        return 0

    lax.fori_loop(0, BPW // 2, step, 0)

    wait(CA, buf_a0, sem_a0)
    wait(CB, buf_b0, sem_b0)
    wait(CA, buf_a1, sem_a1)
    wait(CB, buf_b1, sem_b1)

    pltpu.sync_copy(pooled_v, out_hbm.at[pl.ds(wid * BPW, BPW)])


def _fc_body(x_ref, w_ref, b_ref, o_ref):
    o_ref[...] = (
        jnp.dot(x_ref[...], w_ref[...].T, preferred_element_type=jnp.float32)
        + b_ref[...]
    )


def _fc(pooled, fc_w, fc_b):
    return pl.pallas_call(
        _fc_body,
        out_shape=jax.ShapeDtypeStruct((B, NLAB), jnp.float32),
    )(pooled, fc_w, fc_b.reshape(1, NLAB))


@jax.jit
def kernel(text, emb_table, fc_w, fc_b):
    pooled = _pool_kernel(text.reshape(-1), emb_table)
    return _fc(pooled, fc_w, fc_b)


# speedup vs baseline: 1.0016x; 1.0016x over previous
import functools

import jax
import jax.numpy as jnp
from jax import lax
from jax.experimental import pallas as pl
from jax.experimental.pallas import tpu as pltpu
from jax.experimental.pallas import tpu_sc as plsc

NC = 2
NS = 16
NW = NC * NS
LANE = 16

B = 4096
L = 200
EMB = 64
NLAB = 5
VOCAB_ROWS = 1000000

BPW = B // NW
NIDX = BPW * L
CA, CB = 104, 96
NVEC = EMB // LANE
INV_L = 1.0 / L


def _accum_chunk(buf, n, acc):
    def body(j, acc):
        out = list(acc)
        for u in range(4):
            r = 4 * j + u
            for k in range(NVEC):
                out[k] = out[k] + buf[r, pl.ds(k * LANE, LANE)]
        return tuple(out)

    return lax.fori_loop(0, n // 4, body, acc)


@functools.partial(
    pl.kernel,
    out_type=jax.ShapeDtypeStruct((B, EMB), jnp.float32),
    mesh=plsc.VectorSubcoreMesh(core_axis_name="c", subcore_axis_name="s"),
    compiler_params=pltpu.CompilerParams(use_tc_tiling_on_sc=False),
    scratch_types=[
        pltpu.VMEM((NIDX,), jnp.int32),
        pltpu.VMEM((BPW, EMB), jnp.float32),
        pltpu.VMEM((CA, EMB), jnp.float32),
        pltpu.VMEM((CB, EMB), jnp.float32),
        pltpu.VMEM((CA, EMB), jnp.float32),
        pltpu.VMEM((CB, EMB), jnp.float32),
        pltpu.SemaphoreType.DMA,
        pltpu.SemaphoreType.DMA,
        pltpu.SemaphoreType.DMA,
        pltpu.SemaphoreType.DMA,
    ],
)
def _pool_kernel(idx_hbm, table_hbm, out_hbm, idx_v, pooled_v,
                 buf_a0, buf_b0, buf_a1, buf_b1,
                 sem_a0, sem_b0, sem_a1, sem_b1):
    wid = lax.axis_index("s") * NC + lax.axis_index("c")

    pltpu.sync_copy(idx_hbm.at[pl.ds(wid * NIDX, NIDX)], idx_v)

    def fire(row, off, size, buf, sem):
        start = row * L + off
        pltpu.async_copy(table_hbm.at[idx_v.at[pl.ds(start, size)]], buf, sem)

    def wait(size, buf, sem):
        pltpu.make_async_copy(table_hbm.at[pl.ds(0, size)], buf, sem).wait()

    fire(0, 0, CA, buf_a0, sem_a0)
    fire(0, CA, CB, buf_b0, sem_b0)
    fire(1, 0, CA, buf_a1, sem_a1)
    fire(1, CA, CB, buf_b1, sem_b1)

    zero = jnp.zeros((LANE,), jnp.float32)

    def step(t, carry):
        del carry
        r0 = 2 * t
        r1 = r0 + 1
        n0 = (r0 + 2) & (BPW - 1)
        n1 = (r1 + 2) & (BPW - 1)

        acc = (zero, zero, zero, zero)
        wait(CA, buf_a0, sem_a0)
        acc = _accum_chunk(buf_a0, CA, acc)
        fire(n0, 0, CA, buf_a0, sem_a0)
        wait(CB, buf_b0, sem_b0)
        acc = _accum_chunk(buf_b0, CB, acc)
        fire(n0, CA, CB, buf_b0, sem_b0)
        for k in range(NVEC):
            pooled_v[r0, pl.ds(k * LANE, LANE)] = acc[k] * INV_L

        acc = (zero, zero, zero, zero)
        wait(CA, buf_a1, sem_a1)
        acc = _accum_chunk(buf_a1, CA, acc)
        fire(n1, 0, CA, buf_a1, sem_a1)
        wait(CB, buf_b1, sem_b1)
        acc = _accum_chunk(buf_b1, CB, acc)
        fire(n1, CA, CB, buf_b1, sem_b1)
        for k in range(NVEC):
            pooled_v[r1, pl.ds(k * LANE, LANE)] = acc[k] * INV_L

        return 0

    lax.fori_loop(0, BPW // 2, step, 0)

    wait(CA, buf_a0, sem_a0)
    wait(CB, buf_b0, sem_b0)
    wait(CA, buf_a1, sem_a1)
    wait(CB, buf_b1, sem_b1)

    pltpu.sync_copy(pooled_v, out_hbm.at[pl.ds(wid * BPW, BPW)])


def _fc_body(x_ref, w_ref, b_ref, o_ref):
    o_ref[...] = (
        jnp.dot(x_ref[...], w_ref[...].T, preferred_element_type=jnp.float32)
        + b_ref[...]
    )


def _fc(pooled, fc_w, fc_b):
    return pl.pallas_call(
        _fc_body,
        out_shape=jax.ShapeDtypeStruct((B, NLAB), jnp.float32),
    )(pooled, fc_w, fc_b.reshape(1, NLAB))


@jax.jit
def kernel(text, emb_table, fc_w, fc_b):
    table_lin = jax.lax.optimization_barrier(emb_table.reshape(-1))
    table = table_lin.reshape(VOCAB_ROWS, EMB)
    pooled = _pool_kernel(text.reshape(-1), table)
    return _fc(pooled, fc_w, fc_b)
